# Initial kernel scaffold; baseline (speedup 1.0000x reference)
import jax
import jax.numpy as jnp
from jax.experimental import pallas as pl


def kernel(x, edge_index, edge_attr, Wq, bq, Wk, bk, Wv, bv, We, Wskip, bskip, ln1_g, ln1_b, ln2_g, ln2_b, Wf1, bf1, Wf2, bf2):
    raise NotImplementedError("write your pallas kernel here")


# speedup vs baseline: 8.9632x; 8.9632x over previous
import functools

import jax
import jax.numpy as jnp
from jax import lax
from jax.experimental import pallas as pl
from jax.experimental.pallas import tpu as pltpu
from jax.experimental.pallas import tpu_sc as plsc

N = 10000
E = 320000
D = 128
H = 4
C = 32
ED = 16
DFF = 4 * D

ROW = 128
TQW = 256
NC = 2
NS = 16
HALF = N // NC
AROWS = HALF + 120
CHUNK = 32
NCH = E // CHUNK
CPS = NCH // NS
GROUPS = CHUNK // 16
RPS = AROWS // NS
ZROWS = 16
INV_SQRT_C = 1.0 / (C ** 0.5)

_f32 = jnp.float32
_i32 = jnp.int32


def _pre_body(x_ref, wq_ref, bq_ref, wk_ref, bk_ref, wv_ref, bv_ref, wet_ref,
              wskip_ref, bskip_ref, g1_ref, b1_ref, tq_ref, tkv_ref, skip_ref):
    x = x_ref[...]
    mu = jnp.mean(x, axis=-1, keepdims=True)
    var = jnp.mean((x - mu) ** 2, axis=-1, keepdims=True)
    h = (x - mu) / jnp.sqrt(var + 1e-5) * g1_ref[...] + b1_ref[...]
    q = h @ wq_ref[...] + bq_ref[...]
    k = h @ wk_ref[...] + bk_ref[...]
    v = h @ wv_ref[...] + bv_ref[...]
    wet = wet_ref[...]
    qe = [q[:, i * C:(i + 1) * C] @ wet[i * C:(i + 1) * C, :] for i in range(H)]
    pad = jnp.zeros((x.shape[0], TQW - D - H * ED), _f32)
    tq_ref[...] = jnp.concatenate([q] + qe + [pad], axis=1)
    tkv_ref[...] = jnp.concatenate([k, v], axis=1)
    skip_ref[...] = h @ wskip_ref[...] + bskip_ref[...]


def _sc_body(tq_hbm, tkv_hbm, ea_hbm, src_hbm, dst_hbm, outv_hbm, outs_hbm,
             accv, accs, src_v, dst_v, idx_v, ea_v, kv_v, q_v, pay_a, pay_b,
             zbuf, sem_k, sem_q):
    cid = lax.axis_index("c")
    sid = lax.axis_index("s")
    zeros16 = jnp.zeros((16,), _f32)
    lane = lax.iota(_i32, 16)
    rots = [(lane + sh) & 15 for sh in (8, 4, 2, 1)]
    dst_lo = cid * HALF

    def _zrow(r, carry):
        for j0 in range(0, ROW, 16):
            zbuf[r, pl.ds(j0, 16)] = zeros16
        return carry
    lax.fori_loop(0, ZROWS, _zrow, 0)

    def _zacc(i, carry):
        pltpu.sync_copy(zbuf, accv.at[pl.ds(sid * RPS + i * ZROWS, ZROWS)])
        pltpu.sync_copy(zbuf, accs.at[pl.ds(sid * RPS + i * ZROWS, ZROWS)])
        return carry
    lax.fori_loop(0, RPS // ZROWS, _zacc, 0)

    def _zpay(r, carry):
        for j0 in (80, 96, 112):
            pay_b[r, pl.ds(j0, 16)] = zeros16
        return carry
    lax.fori_loop(0, CHUNK, _zpay, 0)
    plsc.subcore_barrier()

    def _chunk(j, carry):
        base = (sid + NS * j) * CHUNK
        pltpu.sync_copy(src_hbm.at[pl.ds(base, CHUNK)], src_v)
        pltpu.sync_copy(dst_hbm.at[pl.ds(base, CHUNK)], dst_v)
        ck = pltpu.async_copy(tkv_hbm.at[src_v], kv_v, sem_k)
        cq = pltpu.async_copy(tq_hbm.at[dst_v], q_v, sem_q)
        pltpu.sync_copy(ea_hbm.at[pl.ds(base * ED, CHUNK * ED)], ea_v)

        for g in range(GROUPS):
            dv = dst_v[pl.ds(g * 16, 16)] - dst_lo
            ok = (dv >= 0) & (dv < HALF)
            idx_v[pl.ds(g * 16, 16)] = jnp.where(ok, dv, HALF)
        ck.wait()
        cq.wait()

        def _edge(e, carry2):
            ea = ea_v[pl.ds(e * ED, ED)]
            tquad = zeros16
            for i in range(H):
                qk = (q_v[e, pl.ds(i * C, 16)] * kv_v[e, pl.ds(i * C, 16)]
                      + q_v[e, pl.ds(i * C + 16, 16)]
                      * kv_v[e, pl.ds(i * C + 16, 16)]
                      + ea * q_v[e, pl.ds(D + i * ED, 16)])
                for r in rots:
                    qk = qk + qk[r]
                alpha = qk * INV_SQRT_C
                alpha = jnp.minimum(jnp.maximum(alpha, -50.0), 50.0)
                t = jnp.exp(alpha)
                pay_a[e, pl.ds(i * C, 16)] = t * kv_v[e, pl.ds(D + i * C, 16)]
                pay_a[e, pl.ds(i * C + 16, 16)] = (
                    t * kv_v[e, pl.ds(D + i * C + 16, 16)])
                pay_b[e, pl.ds(i * ED, 16)] = t * ea
                tquad = jnp.where(lane == i, t, tquad)
            pay_b[e, pl.ds(H * ED, 16)] = tquad
            return carry2
        lax.fori_loop(0, CHUNK, _edge, 0)
        pltpu.sync_copy(pay_a, accv.at[idx_v], add=True)
        pltpu.sync_copy(pay_b, accs.at[idx_v], add=True)
        return carry
    lax.fori_loop(0, CPS, _chunk, 0)
    plsc.subcore_barrier()

    def _out(i, carry):
        r0 = sid * RPS + i * ZROWS
        pltpu.sync_copy(accv.at[pl.ds(r0, ZROWS)],
                        outv_hbm.at[cid, pl.ds(r0, ZROWS)])
        pltpu.sync_copy(accs.at[pl.ds(r0, ZROWS)],
                        outs_hbm.at[cid, pl.ds(r0, ZROWS)])
        return carry
    lax.fori_loop(0, RPS // ZROWS, _out, 0)


def _post_body(x_ref, skip_ref, accv_ref, accs_ref, we_ref, g2_ref, b2_ref,
               wf1_ref, bf1_ref, wf2_ref, bf2_ref, out_ref):
    accv = accv_ref[...]
    accs = accs_ref[...]
    we = we_ref[...]
    parts = []
    for i in range(H):
        aggv = accv[:, i * C:(i + 1) * C]
        s = accs[:, i * ED:(i + 1) * ED]
        den = accs[:, H * ED + i:H * ED + i + 1]
        parts.append((aggv + s @ we[:, i * C:(i + 1) * C]) / (den + 1e-16))
    conv = jnp.concatenate(parts, axis=1) + skip_ref[...]
    x1 = x_ref[...] + conv
    mu = jnp.mean(x1, axis=-1, keepdims=True)
    var = jnp.mean((x1 - mu) ** 2, axis=-1, keepdims=True)
    h2 = (x1 - mu) / jnp.sqrt(var + 1e-5) * g2_ref[...] + b2_ref[...]
    z = h2 @ wf1_ref[...] + bf1_ref[...]
    gelu = 0.5 * z * (1.0 + lax.erf(z * (2.0 ** -0.5)))
    ffn = gelu @ wf2_ref[...] + bf2_ref[...]
    out_ref[...] = x1 + ffn


_BN = 1000


def _full(shape):
    return pl.BlockSpec(shape, lambda i: tuple(0 for _ in shape))


def _rows(cols):
    return pl.BlockSpec((_BN, cols), lambda i: (i, 0))


@jax.jit
def _run(x, edge_index, edge_attr, Wq, bq, Wk, bk, Wv, bv, We, Wskip, bskip,
         ln1_g, ln1_b, ln2_g, ln2_b, Wf1, bf1, Wf2, bf2):
    src = edge_index[0]
    dst = edge_index[1]
    ea_flat = edge_attr.reshape(-1)
    r2 = lambda a: a.reshape(1, -1)

    tq, tkv, skip = pl.pallas_call(
        _pre_body,
        grid=(N // _BN,),
        in_specs=[_rows(D), _full((D, D)), _full((1, D)), _full((D, D)),
                  _full((1, D)), _full((D, D)), _full((1, D)), _full((D, ED)),
                  _full((D, D)), _full((1, D)), _full((1, D)), _full((1, D))],
        out_specs=[_rows(TQW), _rows(2 * D), _rows(D)],
        out_shape=[jax.ShapeDtypeStruct((N, TQW), _f32),
                   jax.ShapeDtypeStruct((N, 2 * D), _f32),
                   jax.ShapeDtypeStruct((N, D), _f32)],
    )(x, Wq, r2(bq), Wk, r2(bk), Wv, r2(bv), We.T, Wskip, r2(bskip),
      r2(ln1_g), r2(ln1_b))

    sc_edge = functools.partial(
        pl.kernel,
        mesh=plsc.VectorSubcoreMesh(core_axis_name="c", subcore_axis_name="s"),
        out_type=[jax.ShapeDtypeStruct((NC, AROWS, ROW), _f32),
                  jax.ShapeDtypeStruct((NC, AROWS, ROW), _f32)],
        scratch_types=[
            pltpu.VMEM_SHARED((AROWS, ROW), _f32),
            pltpu.VMEM_SHARED((AROWS, ROW), _f32),
            pltpu.VMEM((CHUNK,), _i32),
            pltpu.VMEM((CHUNK,), _i32),
            pltpu.VMEM((CHUNK,), _i32),
            pltpu.VMEM((CHUNK * ED,), _f32),
            pltpu.VMEM((CHUNK, 2 * D), _f32),
            pltpu.VMEM((CHUNK, TQW), _f32),
            pltpu.VMEM((CHUNK, ROW), _f32),
            pltpu.VMEM((CHUNK, ROW), _f32),
            pltpu.VMEM((ZROWS, ROW), _f32),
            pltpu.SemaphoreType.DMA,
            pltpu.SemaphoreType.DMA,
        ],
    )(_sc_body)
    ov, os_ = sc_edge(tq, tkv, ea_flat, src, dst)
    accv = jnp.concatenate([ov[0, :HALF], ov[1, :HALF]], axis=0)
    accse = jnp.concatenate([os_[0, :HALF], os_[1, :HALF]], axis=0)

    out = pl.pallas_call(
        _post_body,
        grid=(N // _BN,),
        in_specs=[_rows(D), _rows(D), _rows(ROW), _rows(ROW), _full((ED, D)),
                  _full((1, D)), _full((1, D)), _full((D, DFF)),
                  _full((1, DFF)), _full((DFF, D)), _full((1, D))],
        out_specs=_rows(D),
        out_shape=jax.ShapeDtypeStruct((N, D), _f32),
    )(x, skip, accv, accse, We, r2(ln2_g), r2(ln2_b), Wf1, r2(bf1),
      Wf2, r2(bf2))
    return out


def kernel(x, edge_index, edge_attr, Wq, bq, Wk, bk, Wv, bv, We, Wskip, bskip,
           ln1_g, ln1_b, ln2_g, ln2_b, Wf1, bf1, Wf2, bf2):
    return _run(x, edge_index, edge_attr, Wq, bq, Wk, bk, Wv, bv, We, Wskip,
                bskip, ln1_g, ln1_b, ln2_g, ln2_b, Wf1, bf1, Wf2, bf2)


# speedup vs baseline: 10.1634x; 1.1339x over previous
import functools

import jax
import jax.numpy as jnp
from jax import lax
from jax.experimental import pallas as pl
from jax.experimental.pallas import tpu as pltpu
from jax.experimental.pallas import tpu_sc as plsc

N = 10000
E = 320000
D = 128
H = 4
C = 32
ED = 16
DFF = 4 * D

ROW = 128
TQW = 256
NC = 2
NS = 16
HALF = N // NC
AROWS = HALF + 120
CHUNK = 32
NCH = E // CHUNK
CPS = NCH // NS
GROUPS = CHUNK // 16
RPS = AROWS // NS
ZROWS = 16
INV_SQRT_C = 1.0 / (C ** 0.5)

_f32 = jnp.float32
_i32 = jnp.int32


def _pre_body(x_ref, wq_ref, bq_ref, wk_ref, bk_ref, wv_ref, bv_ref, wet_ref,
              wskip_ref, bskip_ref, g1_ref, b1_ref, tq_ref, tkv_ref, skip_ref):
    x = x_ref[...]
    mu = jnp.mean(x, axis=-1, keepdims=True)
    var = jnp.mean((x - mu) ** 2, axis=-1, keepdims=True)
    h = (x - mu) / jnp.sqrt(var + 1e-5) * g1_ref[...] + b1_ref[...]
    q = h @ wq_ref[...] + bq_ref[...]
    k = h @ wk_ref[...] + bk_ref[...]
    v = h @ wv_ref[...] + bv_ref[...]
    wet = wet_ref[...]
    qe = [q[:, i * C:(i + 1) * C] @ wet[i * C:(i + 1) * C, :] for i in range(H)]
    pad = jnp.zeros((x.shape[0], TQW - D - H * ED), _f32)
    tq_ref[...] = jnp.concatenate([q] + qe + [pad], axis=1)
    tkv_ref[...] = jnp.concatenate([k, v], axis=1)
    skip_ref[...] = h @ wskip_ref[...] + bskip_ref[...]


def _sc_body(tq_hbm, tkv_hbm, ea_hbm, src_hbm, dst_hbm, outv_hbm, outs_hbm,
             accv, accs, src0, src1, dst0, dst1, idx_v, ea0, ea1,
             kv0, kv1, q0, q1, pay_a, pay_b, zbuf,
             semi0, semi1, semk0, semk1, semq0, semq1):
    cid = lax.axis_index("c")
    sid = lax.axis_index("s")
    zeros16 = jnp.zeros((16,), _f32)
    lane = lax.iota(_i32, 16)
    rots = [(lane + sh) & 15 for sh in (8, 4, 2, 1)]
    dst_lo = cid * HALF
    srcs = (src0, src1)
    dsts = (dst0, dst1)
    eas = (ea0, ea1)
    kvs = (kv0, kv1)
    qs = (q0, q1)
    semis = (semi0, semi1)
    semks = (semk0, semk1)
    semqs = (semq0, semq1)

    def _zrow(r, carry):
        for j0 in range(0, ROW, 16):
            zbuf[r, pl.ds(j0, 16)] = zeros16
        return carry
    lax.fori_loop(0, ZROWS, _zrow, 0)

    def _zacc(i, carry):
        pltpu.sync_copy(zbuf, accv.at[pl.ds(sid * RPS + i * ZROWS, ZROWS)])
        pltpu.sync_copy(zbuf, accs.at[pl.ds(sid * RPS + i * ZROWS, ZROWS)])
        return carry
    lax.fori_loop(0, RPS // ZROWS, _zacc, 0)

    def _zpay(r, carry):
        for j0 in (80, 96, 112):
            pay_b[r, pl.ds(j0, 16)] = zeros16
        return carry
    lax.fori_loop(0, CHUNK, _zpay, 0)
    plsc.subcore_barrier()

    def _issue_idx(j, b):
        base = (sid + NS * j) * CHUNK
        c1 = pltpu.async_copy(src_hbm.at[pl.ds(base, CHUNK)], srcs[b],
                              semis[b])
        c2 = pltpu.async_copy(dst_hbm.at[pl.ds(base, CHUNK)], dsts[b],
                              semis[b])
        c3 = pltpu.async_copy(ea_hbm.at[pl.ds(base * ED, CHUNK * ED)], eas[b],
                              semis[b])
        return (c1, c2, c3)

    def _issue_gather(b):
        ck = pltpu.async_copy(tkv_hbm.at[srcs[b]], kvs[b], semks[b])
        cq = pltpu.async_copy(tq_hbm.at[dsts[b]], qs[b], semqs[b])
        return (ck, cq)

    def _wait(handles):
        for h in handles:
            h.wait()

    def _process(b):
        dst_v = dsts[b]
        ea_v = eas[b]
        kv_v = kvs[b]
        q_v = qs[b]
        for g in range(GROUPS):
            dv = dst_v[pl.ds(g * 16, 16)] - dst_lo
            ok = (dv >= 0) & (dv < HALF)
            idx_v[pl.ds(g * 16, 16)] = jnp.where(ok, dv, HALF)

        def _edge(e, carry2):
            ea = ea_v[pl.ds(e * ED, ED)]
            tquad = zeros16
            for i in range(H):
                qk = (q_v[e, pl.ds(i * C, 16)] * kv_v[e, pl.ds(i * C, 16)]
                      + q_v[e, pl.ds(i * C + 16, 16)]
                      * kv_v[e, pl.ds(i * C + 16, 16)]
                      + ea * q_v[e, pl.ds(D + i * ED, 16)])
                for r in rots:
                    qk = qk + qk[r]
                alpha = qk * INV_SQRT_C
                alpha = jnp.minimum(jnp.maximum(alpha, -50.0), 50.0)
                t = jnp.exp(alpha)
                pay_a[e, pl.ds(i * C, 16)] = t * kv_v[e, pl.ds(D + i * C, 16)]
                pay_a[e, pl.ds(i * C + 16, 16)] = (
                    t * kv_v[e, pl.ds(D + i * C + 16, 16)])
                pay_b[e, pl.ds(i * ED, 16)] = t * ea
                tquad = jnp.where(lane == i, t, tquad)
            pay_b[e, pl.ds(H * ED, 16)] = tquad
            return carry2
        lax.fori_loop(0, CHUNK, _edge, 0)
        pltpu.sync_copy(pay_a, accv.at[idx_v], add=True)
        pltpu.sync_copy(pay_b, accs.at[idx_v], add=True)

    def _pair(p, carry):
        i0 = _issue_idx(2 * p, 0)
        i1 = _issue_idx(2 * p + 1, 1)
        _wait(i0)
        g0 = _issue_gather(0)
        _wait(i1)
        g1 = _issue_gather(1)
        _wait(g0)
        _process(0)
        _wait(g1)
        _process(1)
        return carry
    lax.fori_loop(0, CPS // 2, _pair, 0)
    ilast = _issue_idx(CPS - 1, 0)
    _wait(ilast)
    glast = _issue_gather(0)
    _wait(glast)
    _process(0)
    plsc.subcore_barrier()

    def _out(i, carry):
        r0 = sid * RPS + i * ZROWS
        pltpu.sync_copy(accv.at[pl.ds(r0, ZROWS)],
                        outv_hbm.at[cid, pl.ds(r0, ZROWS)])
        pltpu.sync_copy(accs.at[pl.ds(r0, ZROWS)],
                        outs_hbm.at[cid, pl.ds(r0, ZROWS)])
        return carry
    lax.fori_loop(0, RPS // ZROWS, _out, 0)


def _post_body(x_ref, skip_ref, accv_ref, accs_ref, we_ref, g2_ref, b2_ref,
               wf1_ref, bf1_ref, wf2_ref, bf2_ref, out_ref):
    accv = accv_ref[...]
    accs = accs_ref[...]
    we = we_ref[...]
    parts = []
    for i in range(H):
        aggv = accv[:, i * C:(i + 1) * C]
        s = accs[:, i * ED:(i + 1) * ED]
        den = accs[:, H * ED + i:H * ED + i + 1]
        parts.append((aggv + s @ we[:, i * C:(i + 1) * C]) / (den + 1e-16))
    conv = jnp.concatenate(parts, axis=1) + skip_ref[...]
    x1 = x_ref[...] + conv
    mu = jnp.mean(x1, axis=-1, keepdims=True)
    var = jnp.mean((x1 - mu) ** 2, axis=-1, keepdims=True)
    h2 = (x1 - mu) / jnp.sqrt(var + 1e-5) * g2_ref[...] + b2_ref[...]
    z = h2 @ wf1_ref[...] + bf1_ref[...]
    gelu = 0.5 * z * (1.0 + lax.erf(z * (2.0 ** -0.5)))
    ffn = gelu @ wf2_ref[...] + bf2_ref[...]
    out_ref[...] = x1 + ffn


_BN = 1000


def _full(shape):
    return pl.BlockSpec(shape, lambda i: tuple(0 for _ in shape))


def _rows(cols):
    return pl.BlockSpec((_BN, cols), lambda i: (i, 0))


@jax.jit
def _run(x, edge_index, edge_attr, Wq, bq, Wk, bk, Wv, bv, We, Wskip, bskip,
         ln1_g, ln1_b, ln2_g, ln2_b, Wf1, bf1, Wf2, bf2):
    src = edge_index[0]
    dst = edge_index[1]
    ea_flat = edge_attr.reshape(-1)
    r2 = lambda a: a.reshape(1, -1)

    tq, tkv, skip = pl.pallas_call(
        _pre_body,
        grid=(N // _BN,),
        in_specs=[_rows(D), _full((D, D)), _full((1, D)), _full((D, D)),
                  _full((1, D)), _full((D, D)), _full((1, D)), _full((D, ED)),
                  _full((D, D)), _full((1, D)), _full((1, D)), _full((1, D))],
        out_specs=[_rows(TQW), _rows(2 * D), _rows(D)],
        out_shape=[jax.ShapeDtypeStruct((N, TQW), _f32),
                   jax.ShapeDtypeStruct((N, 2 * D), _f32),
                   jax.ShapeDtypeStruct((N, D), _f32)],
    )(x, Wq, r2(bq), Wk, r2(bk), Wv, r2(bv), We.T, Wskip, r2(bskip),
      r2(ln1_g), r2(ln1_b))

    sc_edge = functools.partial(
        pl.kernel,
        mesh=plsc.VectorSubcoreMesh(core_axis_name="c", subcore_axis_name="s"),
        out_type=[jax.ShapeDtypeStruct((NC, AROWS, ROW), _f32),
                  jax.ShapeDtypeStruct((NC, AROWS, ROW), _f32)],
        scratch_types=[
            pltpu.VMEM_SHARED((AROWS, ROW), _f32),
            pltpu.VMEM_SHARED((AROWS, ROW), _f32),
            pltpu.VMEM((CHUNK,), _i32),
            pltpu.VMEM((CHUNK,), _i32),
            pltpu.VMEM((CHUNK,), _i32),
            pltpu.VMEM((CHUNK,), _i32),
            pltpu.VMEM((CHUNK,), _i32),
            pltpu.VMEM((CHUNK * ED,), _f32),
            pltpu.VMEM((CHUNK * ED,), _f32),
            pltpu.VMEM((CHUNK, 2 * D), _f32),
            pltpu.VMEM((CHUNK, 2 * D), _f32),
            pltpu.VMEM((CHUNK, TQW), _f32),
            pltpu.VMEM((CHUNK, TQW), _f32),
            pltpu.VMEM((CHUNK, ROW), _f32),
            pltpu.VMEM((CHUNK, ROW), _f32),
            pltpu.VMEM((ZROWS, ROW), _f32),
            pltpu.SemaphoreType.DMA,
            pltpu.SemaphoreType.DMA,
            pltpu.SemaphoreType.DMA,
            pltpu.SemaphoreType.DMA,
            pltpu.SemaphoreType.DMA,
            pltpu.SemaphoreType.DMA,
        ],
    )(_sc_body)
    ov, os_ = sc_edge(tq, tkv, ea_flat, src, dst)
    accv = jnp.concatenate([ov[0, :HALF], ov[1, :HALF]], axis=0)
    accse = jnp.concatenate([os_[0, :HALF], os_[1, :HALF]], axis=0)

    out = pl.pallas_call(
        _post_body,
        grid=(N // _BN,),
        in_specs=[_rows(D), _rows(D), _rows(ROW), _rows(ROW), _full((ED, D)),
                  _full((1, D)), _full((1, D)), _full((D, DFF)),
                  _full((1, DFF)), _full((DFF, D)), _full((1, D))],
        out_specs=_rows(D),
        out_shape=jax.ShapeDtypeStruct((N, D), _f32),
    )(x, skip, accv, accse, We, r2(ln2_g), r2(ln2_b), Wf1, r2(bf1),
      Wf2, r2(bf2))
    return out


def kernel(x, edge_index, edge_attr, Wq, bq, Wk, bk, Wv, bv, We, Wskip, bskip,
           ln1_g, ln1_b, ln2_g, ln2_b, Wf1, bf1, Wf2, bf2):
    return _run(x, edge_index, edge_attr, Wq, bq, Wk, bk, Wv, bv, We, Wskip,
                bskip, ln1_g, ln1_b, ln2_g, ln2_b, Wf1, bf1, Wf2, bf2)
